# Initial kernel scaffold; baseline (speedup 1.0000x reference)
import jax
import jax.numpy as jnp
from jax.experimental import pallas as pl


def kernel(query_points, support_points, neighbors, x, bn_weight, bn_bias):
    raise NotImplementedError("write your pallas kernel here")


# speedup vs baseline: 8.2829x; 8.2829x over previous
import functools

import jax
import jax.numpy as jnp
from jax import lax
from jax.experimental import pallas as pl
from jax.experimental.pallas import tpu as pltpu
from jax.experimental.pallas import tpu_sc as plsc

RADIUS = 0.1
INV_R = 1.0 / RADIUS
L = 16
NW = 32
QB = 4
CH = 56
M = 32
TW = 64
D = 48


def _bcast(v, k):
  idx = jnp.full((L, 1), k, dtype=jnp.int32)
  return lax.gather(
      v, idx,
      dimension_numbers=lax.GatherDimensionNumbers(
          offset_dims=(), collapsed_slice_dims=(0,), start_index_map=(0,)),
      slice_sizes=(1,),
      mode=lax.GatherScatterMode.PROMISE_IN_BOUNDS)


def _sc_body(rw, table, nbf, qp, out, idx_c, rows_v, qp_c, out_c, sem):
  c_id = lax.axis_index("c")
  s_id = lax.axis_index("s")
  wid = s_id * 2 + c_id
  row0 = wid * rw

  def chunk_body(c, _):
    base = row0 + c * CH
    pltpu.sync_copy(qp.at[pl.ds(base, CH)], qp_c)
    pltpu.sync_copy(nbf.at[pl.ds(base * M, CH * M)], idx_c)

    def blk_body(q, _):
      pltpu.async_copy(table.at[idx_c.at[pl.ds(q * QB * M, QB * M)]],
                       rows_v, sem).wait()
      for r in range(QB):
        rr = q * QB + r
        qpv = qp_c[rr]
        qx = _bcast(qpv, 0)
        qy = _bcast(qpv, 1)
        qz = _bcast(qpv, 2)
        a0 = jnp.zeros((L,), jnp.float32)
        a1 = jnp.zeros((L,), jnp.float32)
        a2 = jnp.zeros((L,), jnp.float32)
        for m in range(M):
          row = r * M + m
          xyz = rows_v[row, pl.ds(D, L)]
          f0 = rows_v[row, pl.ds(0, L)]
          f1 = rows_v[row, pl.ds(L, L)]
          f2 = rows_v[row, pl.ds(2 * L, L)]
          a0 = a0 + (_bcast(xyz, 0) - qx) * f0
          a1 = a1 + (_bcast(xyz, 1) - qy) * f1
          a2 = a2 + (_bcast(xyz, 2) - qz) * f2
        out_c[rr, pl.ds(0, L)] = a0 * INV_R
        out_c[rr, pl.ds(L, L)] = a1 * INV_R
        out_c[rr, pl.ds(2 * L, L)] = a2 * INV_R
      return ()

    lax.fori_loop(0, CH // QB, blk_body, ())
    pltpu.sync_copy(out_c, out.at[pl.ds(base, CH)])
    return ()

  lax.fori_loop(0, rw // CH, chunk_body, ())


@functools.partial(jax.jit, static_argnums=(3, 4))
def _sc_call(table, nbf, qp16, np_, rw):
  mesh = plsc.VectorSubcoreMesh(core_axis_name="c", subcore_axis_name="s")
  return pl.kernel(
      functools.partial(_sc_body, rw),
      out_type=jax.ShapeDtypeStruct((np_, D), jnp.float32),
      mesh=mesh,
      scratch_types=[
          pltpu.VMEM((CH * M,), jnp.int32),
          pltpu.VMEM((QB * M, TW), jnp.float32),
          pltpu.VMEM((CH, 16), jnp.float32),
          pltpu.VMEM((CH, D), jnp.float32),
          pltpu.SemaphoreType.DMA,
      ],
      compiler_params=pltpu.CompilerParams(use_tc_tiling_on_sc=False),
  )(table, nbf, qp16)


def _max_body(nb_ref, out_ref):
  i = pl.program_id(0)

  @pl.when(i == 0)
  def _():
    out_ref[0, 0] = jnp.int32(-2**31)

  out_ref[0, 0] = jnp.maximum(out_ref[0, 0], jnp.max(nb_ref[...]))


def _divstats_body(pm_ref, raw_ref, nb_ref, a_ref, st_ref):
  i = pl.program_id(0)
  pm = pm_ref[0, 0]
  cnt = jnp.sum((nb_ref[...] < pm).astype(jnp.float32), axis=1,
                keepdims=True) + 1e-5
  a = raw_ref[...] / cnt
  a_ref[...] = a

  @pl.when(i == 0)
  def _():
    st_ref[...] = jnp.zeros_like(st_ref)

  st_ref[0:1, :] += jnp.sum(a, axis=0, keepdims=True)
  st_ref[1:2, :] += jnp.sum(a * a, axis=0, keepdims=True)


def _norm_body(n_rows, st_ref, w_ref, b_ref, a_ref, o_ref):
  mean = st_ref[0:1, :] / n_rows
  var = st_ref[1:2, :] / n_rows - mean * mean
  inv = lax.rsqrt(var + 1e-5)
  y = (a_ref[...] - mean) * inv * w_ref[...] + b_ref[...]
  o_ref[...] = jnp.maximum(y, 0.2 * y)


def kernel(query_points, support_points, neighbors, x, bn_weight, bn_bias):
  n, d = x.shape
  assert d == D
  nb = neighbors.astype(jnp.int32)

  table = jnp.concatenate(
      [x, support_points,
       jnp.zeros((n, TW - D - 3), jnp.float32)], axis=1)
  np_ = ((n + NW * CH - 1) // (NW * CH)) * (NW * CH)
  rw = np_ // NW
  nbf = jnp.pad(nb, ((0, np_ - n), (0, 0))).reshape(-1)
  qp16 = jnp.pad(query_points, ((0, np_ - n), (0, 16 - 3)))

  agg_raw = _sc_call(table, nbf, qp16, np_, rw)[:n]

  blk = 2000
  grid = n // blk
  padmax = pl.pallas_call(
      _max_body,
      grid=(grid,),
      in_specs=[pl.BlockSpec((blk, M), lambda i: (i, 0))],
      out_specs=pl.BlockSpec((1, 1), lambda i: (0, 0),
                             memory_space=pltpu.SMEM),
      out_shape=jax.ShapeDtypeStruct((1, 1), jnp.int32),
  )(nb)

  a, st = pl.pallas_call(
      _divstats_body,
      grid=(grid,),
      in_specs=[
          pl.BlockSpec((1, 1), lambda i: (0, 0), memory_space=pltpu.SMEM),
          pl.BlockSpec((blk, D), lambda i: (i, 0)),
          pl.BlockSpec((blk, M), lambda i: (i, 0)),
      ],
      out_specs=[
          pl.BlockSpec((blk, D), lambda i: (i, 0)),
          pl.BlockSpec((2, D), lambda i: (0, 0)),
      ],
      out_shape=[
          jax.ShapeDtypeStruct((n, D), jnp.float32),
          jax.ShapeDtypeStruct((2, D), jnp.float32),
      ],
  )(padmax, agg_raw, nb)

  out = pl.pallas_call(
      functools.partial(_norm_body, float(n)),
      grid=(grid,),
      in_specs=[
          pl.BlockSpec((2, D), lambda i: (0, 0)),
          pl.BlockSpec((1, D), lambda i: (0, 0)),
          pl.BlockSpec((1, D), lambda i: (0, 0)),
          pl.BlockSpec((blk, D), lambda i: (i, 0)),
      ],
      out_specs=pl.BlockSpec((blk, D), lambda i: (i, 0)),
      out_shape=jax.ShapeDtypeStruct((n, D), jnp.float32),
  )(st, bn_weight.reshape(1, D), bn_bias.reshape(1, D), a)

  return out


# speedup vs baseline: 11.3839x; 1.3744x over previous
import functools

import jax
import jax.numpy as jnp
from jax import lax
from jax.experimental import pallas as pl
from jax.experimental.pallas import tpu as pltpu
from jax.experimental.pallas import tpu_sc as plsc

RADIUS = 0.1
INV_R = 1.0 / RADIUS
L = 16
NW = 32
QB = 4
CH = 112
M = 32
TW = 64
D = 48


def _bcast(v, k):
  idx = jnp.full((L, 1), k, dtype=jnp.int32)
  return lax.gather(
      v, idx,
      dimension_numbers=lax.GatherDimensionNumbers(
          offset_dims=(), collapsed_slice_dims=(0,), start_index_map=(0,)),
      slice_sizes=(1,),
      mode=lax.GatherScatterMode.PROMISE_IN_BOUNDS)


def _sc_body(rw, table, nbf, qp, out, idx_a, qp_a, rows_a, rows_b, out_c,
             sem_a, sem_b):
  c_id = lax.axis_index("c")
  s_id = lax.axis_index("s")
  wid = s_id * 2 + c_id
  row0 = wid * rw
  nblk = rw // QB
  bc = CH // QB

  pltpu.sync_copy(nbf.at[pl.ds(row0 * M, rw * M)], idx_a)
  pltpu.sync_copy(qp.at[pl.ds(row0, rw)], qp_a)

  def gather(b, buf, sem):
    pltpu.async_copy(table.at[idx_a.at[pl.ds(b * QB * M, QB * M)]], buf, sem)

  def drain(buf, sem):
    pltpu.make_async_copy(table.at[idx_a.at[pl.ds(0, QB * M)]], buf,
                          sem).wait()

  def compute(b, buf):
    ob = lax.rem(b, bc) * QB
    for r in range(QB):
      rr = b * QB + r
      orow = ob + r
      qpv = qp_a[rr]
      qx = _bcast(qpv, 0)
      qy = _bcast(qpv, 1)
      qz = _bcast(qpv, 2)
      a0 = jnp.zeros((L,), jnp.float32)
      a1 = jnp.zeros((L,), jnp.float32)
      a2 = jnp.zeros((L,), jnp.float32)
      for m in range(M):
        row = r * M + m
        xyz = buf[row, pl.ds(D, L)]
        f0 = buf[row, pl.ds(0, L)]
        f1 = buf[row, pl.ds(L, L)]
        f2 = buf[row, pl.ds(2 * L, L)]
        a0 = a0 + (_bcast(xyz, 0) - qx) * f0
        a1 = a1 + (_bcast(xyz, 1) - qy) * f1
        a2 = a2 + (_bcast(xyz, 2) - qz) * f2
      out_c[orow, pl.ds(0, L)] = a0 * INV_R
      out_c[orow, pl.ds(L, L)] = a1 * INV_R
      out_c[orow, pl.ds(2 * L, L)] = a2 * INV_R

  gather(0, rows_a, sem_a)

  def body(i, _):
    b0 = 2 * i
    b1 = b0 + 1
    gather(b1, rows_b, sem_b)
    drain(rows_a, sem_a)
    compute(b0, rows_a)

    @pl.when(b1 + 1 < nblk)
    def _():
      gather(b1 + 1, rows_a, sem_a)

    drain(rows_b, sem_b)
    compute(b1, rows_b)

    @pl.when(lax.rem(b1, bc) == bc - 1)
    def _():
      pltpu.sync_copy(out_c,
                      out.at[pl.ds(row0 + lax.div(b1, bc) * CH, CH)])

    return ()

  lax.fori_loop(0, nblk // 2, body, ())


@functools.partial(jax.jit, static_argnums=(3, 4))
def _sc_call(table, nbf, qp16, np_, rw):
  mesh = plsc.VectorSubcoreMesh(core_axis_name="c", subcore_axis_name="s")
  return pl.kernel(
      functools.partial(_sc_body, rw),
      out_type=jax.ShapeDtypeStruct((np_, D), jnp.float32),
      mesh=mesh,
      scratch_types=[
          pltpu.VMEM((rw * M,), jnp.int32),
          pltpu.VMEM((rw, 16), jnp.float32),
          pltpu.VMEM((QB * M, TW), jnp.float32),
          pltpu.VMEM((QB * M, TW), jnp.float32),
          pltpu.VMEM((CH, D), jnp.float32),
          pltpu.SemaphoreType.DMA,
          pltpu.SemaphoreType.DMA,
      ],
      compiler_params=pltpu.CompilerParams(use_tc_tiling_on_sc=False),
  )(table, nbf, qp16)


def _max_body(nb_ref, out_ref):
  i = pl.program_id(0)

  @pl.when(i == 0)
  def _():
    out_ref[0, 0] = jnp.int32(-2**31)

  out_ref[0, 0] = jnp.maximum(out_ref[0, 0], jnp.max(nb_ref[...]))


def _divstats_body(pm_ref, raw_ref, nb_ref, a_ref, st_ref):
  i = pl.program_id(0)
  pm = pm_ref[0, 0]
  cnt = jnp.sum((nb_ref[...] < pm).astype(jnp.float32), axis=1,
                keepdims=True) + 1e-5
  a = raw_ref[...] / cnt
  a_ref[...] = a

  @pl.when(i == 0)
  def _():
    st_ref[...] = jnp.zeros_like(st_ref)

  st_ref[0:1, :] += jnp.sum(a, axis=0, keepdims=True)
  st_ref[1:2, :] += jnp.sum(a * a, axis=0, keepdims=True)


def _norm_body(n_rows, st_ref, w_ref, b_ref, a_ref, o_ref):
  mean = st_ref[0:1, :] / n_rows
  var = st_ref[1:2, :] / n_rows - mean * mean
  inv = lax.rsqrt(var + 1e-5)
  y = (a_ref[...] - mean) * inv * w_ref[...] + b_ref[...]
  o_ref[...] = jnp.maximum(y, 0.2 * y)


def kernel(query_points, support_points, neighbors, x, bn_weight, bn_bias):
  n, d = x.shape
  assert d == D
  nb = neighbors.astype(jnp.int32)

  table = jnp.concatenate(
      [x, support_points,
       jnp.zeros((n, TW - D - 3), jnp.float32)], axis=1)
  np_ = ((n + NW * CH - 1) // (NW * CH)) * (NW * CH)
  rw = np_ // NW
  nbf = jnp.pad(nb, ((0, np_ - n), (0, 0))).reshape(-1)
  qp16 = jnp.pad(query_points, ((0, np_ - n), (0, 16 - 3)))

  agg_raw = _sc_call(table, nbf, qp16, np_, rw)[:n]

  blk = 2000
  grid = n // blk
  padmax = pl.pallas_call(
      _max_body,
      grid=(grid,),
      in_specs=[pl.BlockSpec((blk, M), lambda i: (i, 0))],
      out_specs=pl.BlockSpec((1, 1), lambda i: (0, 0),
                             memory_space=pltpu.SMEM),
      out_shape=jax.ShapeDtypeStruct((1, 1), jnp.int32),
  )(nb)

  a, st = pl.pallas_call(
      _divstats_body,
      grid=(grid,),
      in_specs=[
          pl.BlockSpec((1, 1), lambda i: (0, 0), memory_space=pltpu.SMEM),
          pl.BlockSpec((blk, D), lambda i: (i, 0)),
          pl.BlockSpec((blk, M), lambda i: (i, 0)),
      ],
      out_specs=[
          pl.BlockSpec((blk, D), lambda i: (i, 0)),
          pl.BlockSpec((2, D), lambda i: (0, 0)),
      ],
      out_shape=[
          jax.ShapeDtypeStruct((n, D), jnp.float32),
          jax.ShapeDtypeStruct((2, D), jnp.float32),
      ],
  )(padmax, agg_raw, nb)

  out = pl.pallas_call(
      functools.partial(_norm_body, float(n)),
      grid=(grid,),
      in_specs=[
          pl.BlockSpec((2, D), lambda i: (0, 0)),
          pl.BlockSpec((1, D), lambda i: (0, 0)),
          pl.BlockSpec((1, D), lambda i: (0, 0)),
          pl.BlockSpec((blk, D), lambda i: (i, 0)),
      ],
      out_specs=pl.BlockSpec((blk, D), lambda i: (i, 0)),
      out_shape=jax.ShapeDtypeStruct((n, D), jnp.float32),
  )(st, bn_weight.reshape(1, D), bn_bias.reshape(1, D), a)

  return out
